# baseline (speedup 1.0000x reference)
import functools

import jax
import jax.numpy as jnp
from jax import lax
from jax.experimental import pallas as pl
from jax.experimental.pallas import tpu as pltpu
from jax.experimental.pallas import tpu_sc as plsc

NUM_EMB = 1000000
DIM = 64
B, T = 4096, 50
TOTAL = B * T
NC, NS = 2, 16
NW = NC * NS
PER_W = TOTAL // NW
CHUNK = 128
N_CHUNKS = PER_W // CHUNK
NBUF = 5


def _sc_gather(idx3d, weight):
  mesh = plsc.VectorSubcoreMesh(core_axis_name="c", subcore_axis_name="s")

  @functools.partial(
      pl.kernel,
      mesh=mesh,
      out_type=jax.ShapeDtypeStruct((TOTAL, DIM), jnp.float32),
      compiler_params=pltpu.CompilerParams(use_tc_tiling_on_sc=False),
      scratch_types=[
          pltpu.VMEM((N_CHUNKS, CHUNK), jnp.int32),
      ] + [pltpu.VMEM((CHUNK, DIM), jnp.float32) for _ in range(NBUF)]
        + [pltpu.SemaphoreType.DMA for _ in range(NBUF)],
  )
  def k(idx_hbm, table_hbm, out_hbm, idx_v, *bufs_and_sems):
    rows = bufs_and_sems[:NBUF]
    sems = bufs_and_sems[NBUF:]
    wid = lax.axis_index("s") * NC + lax.axis_index("c")
    base = wid * PER_W
    pltpu.sync_copy(idx_hbm.at[wid], idx_v)

    for p in range(NBUF):
      pltpu.async_copy(table_hbm.at[idx_v.at[p]], rows[p], sems[p])

    def body(g, carry):
      for p in range(NBUF):
        j = NBUF * g + p
        buf, sem = rows[p], sems[p]
        pltpu.make_async_copy(table_hbm.at[idx_v.at[0]], buf, sem).wait()
        pltpu.sync_copy(buf, out_hbm.at[pl.ds(base + j * CHUNK, CHUNK)])

        @pl.when(j + NBUF < N_CHUNKS)
        def _():
          pltpu.async_copy(table_hbm.at[idx_v.at[j + NBUF]], buf, sem)

      return carry

    lax.fori_loop(0, N_CHUNKS // NBUF, body, 0)

  return k(idx3d, weight)


TN = 512


def _tc_transpose(wt):
  def body(in_ref, out_ref):
    x = in_ref[...]
    xg = jnp.concatenate(
        [x[:, g * TN:(g + 1) * TN] for g in range(4)], axis=0)
    ii = lax.broadcasted_iota(jnp.int32, (4 * DIM, 4 * DIM), 0)
    jj = lax.broadcasted_iota(jnp.int32, (4 * DIM, 4 * DIM), 1)
    eye = (ii == jj).astype(jnp.float32)
    out_ref[...] = lax.dot_general(
        xg, eye, (((0,), (0,)), ((), ())),
        precision=lax.Precision.HIGHEST,
        preferred_element_type=jnp.float32)

  grid = (NUM_EMB + 4 * TN - 1) // (4 * TN)
  return pl.pallas_call(
      body,
      grid=(grid,),
      in_specs=[pl.BlockSpec((DIM, 4 * TN), lambda i: (0, i))],
      out_specs=pl.BlockSpec((TN, 4 * DIM), lambda i: (i, 0)),
      out_shape=jax.ShapeDtypeStruct((grid * TN, 4 * DIM), jnp.float32),
  )(wt)


def kernel(IX, weight):
  wt_pairs = _tc_transpose(weight.T)
  wt_rows = wt_pairs.reshape(-1, DIM)
  ix = IX.astype(jnp.int32)
  ixm = 4 * ((ix // (4 * TN)) * TN + ix % TN) + (ix // TN) % 4
  idx3d = ixm.reshape(NW, N_CHUNKS, CHUNK)
  out = _sc_gather(idx3d, wt_rows)
  return out.reshape(B, T, DIM)


# speedup vs baseline: 1.0445x; 1.0445x over previous
import functools

import jax
import jax.numpy as jnp
from jax import lax
from jax.experimental import pallas as pl
from jax.experimental.pallas import tpu as pltpu
from jax.experimental.pallas import tpu_sc as plsc

NUM_EMB = 1000000
DIM = 64
B, T = 4096, 50
TOTAL = B * T
NC, NS = 2, 16
NW = NC * NS
PER_W = TOTAL // NW
CHUNK = 128
N_CHUNKS = PER_W // CHUNK
NBUF = 5


def _sc_gather(idx3d, weight):
  mesh = plsc.VectorSubcoreMesh(core_axis_name="c", subcore_axis_name="s")

  @functools.partial(
      pl.kernel,
      mesh=mesh,
      out_type=jax.ShapeDtypeStruct((TOTAL, DIM), jnp.float32),
      compiler_params=pltpu.CompilerParams(use_tc_tiling_on_sc=False),
      scratch_types=[
          pltpu.VMEM((N_CHUNKS, CHUNK), jnp.int32),
      ] + [pltpu.VMEM((CHUNK, DIM), jnp.float32) for _ in range(NBUF)]
        + [pltpu.SemaphoreType.DMA for _ in range(NBUF)],
  )
  def k(idx_hbm, table_hbm, out_hbm, idx_v, *bufs_and_sems):
    rows = bufs_and_sems[:NBUF]
    sems = bufs_and_sems[NBUF:]
    wid = lax.axis_index("s") * NC + lax.axis_index("c")
    base = wid * PER_W
    pltpu.sync_copy(idx_hbm.at[wid], idx_v)

    for p in range(NBUF):
      pltpu.async_copy(table_hbm.at[idx_v.at[p]], rows[p], sems[p])

    def body(g, carry):
      for p in range(NBUF):
        j = NBUF * g + p
        buf, sem = rows[p], sems[p]
        pltpu.make_async_copy(table_hbm.at[idx_v.at[0]], buf, sem).wait()
        pltpu.sync_copy(buf, out_hbm.at[pl.ds(base + j * CHUNK, CHUNK)])

        @pl.when(j + NBUF < N_CHUNKS)
        def _():
          pltpu.async_copy(table_hbm.at[idx_v.at[j + NBUF]], buf, sem)

      return carry

    lax.fori_loop(0, N_CHUNKS // NBUF, body, 0)

  return k(idx3d, weight)


TN = 512


def _tc_transpose(wt):
  def body(in_ref, out_ref):
    x = in_ref[...]
    xg = jnp.concatenate(
        [x[:, g * TN:(g + 1) * TN] for g in range(4)], axis=0)
    ii = lax.broadcasted_iota(jnp.int32, (4 * DIM, 4 * DIM), 0)
    jj = lax.broadcasted_iota(jnp.int32, (4 * DIM, 4 * DIM), 1)
    eye = (ii == jj).astype(jnp.float32)

    def mm(v):
      return lax.dot_general(
          v, eye, (((0,), (0,)), ((), ())),
          preferred_element_type=jnp.float32)

    h1 = xg.astype(jnp.bfloat16).astype(jnp.float32)
    r1 = xg - h1
    h2 = r1.astype(jnp.bfloat16).astype(jnp.float32)
    r2 = r1 - h2
    out_ref[...] = mm(h1) + mm(h2) + mm(r2)

  grid = (NUM_EMB + 4 * TN - 1) // (4 * TN)
  return pl.pallas_call(
      body,
      grid=(grid,),
      in_specs=[pl.BlockSpec((DIM, 4 * TN), lambda i: (0, i))],
      out_specs=pl.BlockSpec((TN, 4 * DIM), lambda i: (i, 0)),
      out_shape=jax.ShapeDtypeStruct((grid * TN, 4 * DIM), jnp.float32),
  )(wt)


def kernel(IX, weight):
  wt_pairs = _tc_transpose(weight.T)
  wt_rows = wt_pairs.reshape(-1, DIM)
  ix = IX.astype(jnp.int32)
  ixm = 4 * ((ix // (4 * TN)) * TN + ix % TN) + (ix // TN) % 4
  idx3d = ixm.reshape(NW, N_CHUNKS, CHUNK)
  out = _sc_gather(idx3d, wt_rows)
  return out.reshape(B, T, DIM)


# speedup vs baseline: 1.9064x; 1.8252x over previous
import functools

import jax
import jax.numpy as jnp
from jax import lax
from jax.experimental import pallas as pl
from jax.experimental.pallas import tpu as pltpu
from jax.experimental.pallas import tpu_sc as plsc

NUM_EMB = 1000000
DIM = 64
B, T = 4096, 50
TOTAL = B * T
NC, NS = 2, 16
NW = NC * NS
PER_W = TOTAL // NW
CHUNK = 128
N_CHUNKS = PER_W // CHUNK
NBUF = 5


def _sc_gather(idx3d, weight):
  mesh = plsc.VectorSubcoreMesh(core_axis_name="c", subcore_axis_name="s")

  @functools.partial(
      pl.kernel,
      mesh=mesh,
      out_type=jax.ShapeDtypeStruct((TOTAL, DIM), jnp.float32),
      compiler_params=pltpu.CompilerParams(use_tc_tiling_on_sc=False),
      scratch_types=[
          pltpu.VMEM((N_CHUNKS, CHUNK), jnp.int32),
      ] + [pltpu.VMEM((CHUNK, DIM), jnp.float32) for _ in range(NBUF)]
        + [pltpu.SemaphoreType.DMA for _ in range(NBUF)],
  )
  def k(idx_hbm, table_hbm, out_hbm, idx_v, *bufs_and_sems):
    rows = bufs_and_sems[:NBUF]
    sems = bufs_and_sems[NBUF:]
    wid = lax.axis_index("s") * NC + lax.axis_index("c")
    base = wid * PER_W
    pltpu.sync_copy(idx_hbm.at[wid], idx_v)

    for p in range(NBUF):
      pltpu.async_copy(table_hbm.at[idx_v.at[p]], rows[p], sems[p])

    def body(g, carry):
      for p in range(NBUF):
        j = NBUF * g + p
        buf, sem = rows[p], sems[p]
        pltpu.make_async_copy(table_hbm.at[idx_v.at[0]], buf, sem).wait()
        pltpu.sync_copy(buf, out_hbm.at[pl.ds(base + j * CHUNK, CHUNK)])

        @pl.when(j + NBUF < N_CHUNKS)
        def _():
          pltpu.async_copy(table_hbm.at[idx_v.at[j + NBUF]], buf, sem)

      return carry

    lax.fori_loop(0, N_CHUNKS // NBUF, body, 0)

  return k(idx3d, weight)


TN = 8192


def _tc_transpose(wt):
  def body(in_ref, out_ref):
    x = in_ref[...]
    out_ref[:, :DIM] = jnp.transpose(x[:, :TN // 2], (1, 0))
    out_ref[:, DIM:] = jnp.transpose(x[:, TN // 2:], (1, 0))

  grid = (NUM_EMB + TN - 1) // TN
  n_rows = grid * (TN // 2)
  return pl.pallas_call(
      body,
      grid=(grid,),
      in_specs=[pl.BlockSpec((DIM, TN), lambda i: (0, i))],
      out_specs=pl.BlockSpec((TN // 2, 2 * DIM), lambda i: (i, 0)),
      out_shape=jax.ShapeDtypeStruct((n_rows, 2 * DIM), jnp.float32),
  )(wt)


def kernel(IX, weight):
  wt_pairs = _tc_transpose(weight.T)
  wt_rows = wt_pairs.reshape(-1, DIM)
  ix = IX.astype(jnp.int32)
  c = ix % TN
  ixm = (ix // TN) * TN + 2 * (c % (TN // 2)) + c // (TN // 2)
  idx3d = ixm.reshape(NW, N_CHUNKS, CHUNK)
  out = _sc_gather(idx3d, wt_rows)
  return out.reshape(B, T, DIM)


# speedup vs baseline: 2.0551x; 1.0780x over previous
import functools

import jax
import jax.numpy as jnp
from jax import lax
from jax.experimental import pallas as pl
from jax.experimental.pallas import tpu as pltpu
from jax.experimental.pallas import tpu_sc as plsc

NUM_EMB = 1000000
DIM = 64
B, T = 4096, 50
TOTAL = B * T
NC, NS = 2, 16
NW = NC * NS
PER_W = TOTAL // NW
CHUNK = 128
N_CHUNKS = PER_W // CHUNK
NBUF = 5


def _sc_gather(idx3d, weight):
  mesh = plsc.VectorSubcoreMesh(core_axis_name="c", subcore_axis_name="s")

  @functools.partial(
      pl.kernel,
      mesh=mesh,
      out_type=jax.ShapeDtypeStruct((TOTAL, DIM), jnp.float32),
      compiler_params=pltpu.CompilerParams(use_tc_tiling_on_sc=False),
      scratch_types=[
          pltpu.VMEM((N_CHUNKS, CHUNK), jnp.int32),
      ] + [pltpu.VMEM((CHUNK, DIM), jnp.float32) for _ in range(NBUF)]
        + [pltpu.SemaphoreType.DMA for _ in range(NBUF)],
  )
  def k(idx_hbm, table_hbm, out_hbm, idx_v, *bufs_and_sems):
    rows = bufs_and_sems[:NBUF]
    sems = bufs_and_sems[NBUF:]
    wid = lax.axis_index("s") * NC + lax.axis_index("c")
    base = wid * PER_W
    pltpu.sync_copy(idx_hbm.at[wid], idx_v)

    for p in range(NBUF):
      pltpu.async_copy(table_hbm.at[idx_v.at[p]], rows[p], sems[p])

    def body(g, carry):
      for p in range(NBUF):
        j = NBUF * g + p
        buf, sem = rows[p], sems[p]
        pltpu.make_async_copy(table_hbm.at[idx_v.at[0]], buf, sem).wait()
        pltpu.sync_copy(buf, out_hbm.at[pl.ds(base + j * CHUNK, CHUNK)])

        @pl.when(j + NBUF < N_CHUNKS)
        def _():
          pltpu.async_copy(table_hbm.at[idx_v.at[j + NBUF]], buf, sem)

      return carry

    lax.fori_loop(0, N_CHUNKS // NBUF, body, 0)

  return k(idx3d, weight)


TN = 16384


def _tc_transpose(wt):
  def body(in_ref, out_ref):
    x = in_ref[...]
    out_ref[:, :DIM] = jnp.transpose(x[:, :TN // 2], (1, 0))
    out_ref[:, DIM:] = jnp.transpose(x[:, TN // 2:], (1, 0))

  grid = (NUM_EMB + TN - 1) // TN
  n_rows = grid * (TN // 2)
  return pl.pallas_call(
      body,
      grid=(grid,),
      in_specs=[pl.BlockSpec((DIM, TN), lambda i: (0, i))],
      out_specs=pl.BlockSpec((TN // 2, 2 * DIM), lambda i: (i, 0)),
      out_shape=jax.ShapeDtypeStruct((n_rows, 2 * DIM), jnp.float32),
  )(wt)


def kernel(IX, weight):
  wt_pairs = _tc_transpose(weight.T)
  wt_rows = wt_pairs.reshape(-1, DIM)
  ix = IX.astype(jnp.int32)
  c = ix % TN
  ixm = (ix // TN) * TN + 2 * (c % (TN // 2)) + c // (TN // 2)
  idx3d = ixm.reshape(NW, N_CHUNKS, CHUNK)
  out = _sc_gather(idx3d, wt_rows)
  return out.reshape(B, T, DIM)


# speedup vs baseline: 2.1319x; 1.0374x over previous
import functools

import jax
import jax.numpy as jnp
from jax import lax
from jax.experimental import pallas as pl
from jax.experimental.pallas import tpu as pltpu
from jax.experimental.pallas import tpu_sc as plsc

NUM_EMB = 1000000
DIM = 64
B, T = 4096, 50
TOTAL = B * T
NC, NS = 2, 16
NW = NC * NS
PER_W = TOTAL // NW
CHUNK = 128
N_CHUNKS = PER_W // CHUNK
NBUF = 5


def _sc_gather(idx3d, weight):
  mesh = plsc.VectorSubcoreMesh(core_axis_name="c", subcore_axis_name="s")

  @functools.partial(
      pl.kernel,
      mesh=mesh,
      out_type=jax.ShapeDtypeStruct((TOTAL, DIM), jnp.float32),
      compiler_params=pltpu.CompilerParams(use_tc_tiling_on_sc=False),
      scratch_types=[
          pltpu.VMEM((N_CHUNKS, CHUNK), jnp.int32),
      ] + [pltpu.VMEM((CHUNK, DIM), jnp.float32) for _ in range(NBUF)]
        + [pltpu.SemaphoreType.DMA for _ in range(NBUF)],
  )
  def k(idx_hbm, table_hbm, out_hbm, idx_v, *bufs_and_sems):
    rows = bufs_and_sems[:NBUF]
    sems = bufs_and_sems[NBUF:]
    wid = lax.axis_index("s") * NC + lax.axis_index("c")
    base = wid * PER_W
    pltpu.sync_copy(idx_hbm.at[wid], idx_v)

    for p in range(NBUF):
      pltpu.async_copy(table_hbm.at[idx_v.at[p]], rows[p], sems[p])

    def body(g, carry):
      for p in range(NBUF):
        j = NBUF * g + p
        buf, sem = rows[p], sems[p]
        pltpu.make_async_copy(table_hbm.at[idx_v.at[0]], buf, sem).wait()
        pltpu.sync_copy(buf, out_hbm.at[pl.ds(base + j * CHUNK, CHUNK)])

        @pl.when(j + NBUF < N_CHUNKS)
        def _():
          pltpu.async_copy(table_hbm.at[idx_v.at[j + NBUF]], buf, sem)

      return carry

    lax.fori_loop(0, N_CHUNKS // NBUF, body, 0)

  return k(idx3d, weight)


TN = 32768


def _tc_transpose(wt):
  def body(in_ref, out_ref):
    x = in_ref[...]
    out_ref[:, :DIM] = jnp.transpose(x[:, :TN // 2], (1, 0))
    out_ref[:, DIM:] = jnp.transpose(x[:, TN // 2:], (1, 0))

  grid = (NUM_EMB + TN - 1) // TN
  n_rows = grid * (TN // 2)
  return pl.pallas_call(
      body,
      grid=(grid,),
      in_specs=[pl.BlockSpec((DIM, TN), lambda i: (0, i))],
      out_specs=pl.BlockSpec((TN // 2, 2 * DIM), lambda i: (i, 0)),
      out_shape=jax.ShapeDtypeStruct((n_rows, 2 * DIM), jnp.float32),
  )(wt)


def kernel(IX, weight):
  wt_pairs = _tc_transpose(weight.T)
  wt_rows = wt_pairs.reshape(-1, DIM)
  ix = IX.astype(jnp.int32)
  c = ix % TN
  ixm = (ix // TN) * TN + 2 * (c % (TN // 2)) + c // (TN // 2)
  idx3d = ixm.reshape(NW, N_CHUNKS, CHUNK)
  out = _sc_gather(idx3d, wt_rows)
  return out.reshape(B, T, DIM)
